# baseline (speedup 1.0000x reference)
import functools
import math

import jax
import jax.numpy as jnp
from jax import lax
from jax.experimental import pallas as pl
from jax.experimental.pallas import tpu as pltpu
from jax.experimental.pallas import tpu_sc as plsc

NC = 2
NS = 16
L = 16
NW = NC * NS


def _mesh():
    return plsc.VectorSubcoreMesh(
        core_axis_name="c", subcore_axis_name="s",
        num_cores=NC, num_subcores=NS)


def _wid():
    return lax.axis_index("s") * NC + lax.axis_index("c")


def _iota16():
    return lax.iota(jnp.int32, 16)


def _transpose_block(src, dst, rows, scale):
    @pl.loop(0, rows, unroll=4)
    def _row(r):
        for k in range(8):
            rowv = _iota16() + 16 * (k % 4)
            colv = jnp.full((16,), 2 * r + (1 if k >= 4 else 0), jnp.int32)
            vals = plsc.load_gather(src, [rowv, colv]) * scale
            dst[r, pl.ds(16 * k, 16)] = vals


@functools.lru_cache(maxsize=None)
def _phase1(vocab: int, d: int):
    assert d == 64
    nblk = vocab // 128
    tail = vocab % 128
    assert tail in (0, 64)
    nfull = nblk // NW
    extra = nblk % NW
    scale = math.sqrt(d)

    def body(tt, tp, inb, outb, int_, outt, *sems):
        gs, ss = sems[:2], sems[2:]
        wid = _wid()

        def fire_in(t, b):
            return pltpu.make_async_copy(
                tt.at[:, pl.ds((wid + NW * t) * 128, 128)], inb.at[b], gs[b])

        def fire_out(t, b):
            return pltpu.make_async_copy(
                outb.at[b], tp.at[pl.ds((wid + NW * t) * 64, 64)], ss[b])

        fire_in(0, 0).start()

        @pl.loop(0, nfull // 2)
        def _grp(g):
            for i in range(2):
                t = 2 * g + i
                @pl.when(t + 1 < nfull)
                def _():
                    fire_in(t + 1, (i + 1) % 2).start()
                fire_in(t, i).wait()
                @pl.when(t >= 2)
                def _():
                    fire_out(t - 2, i).wait()
                _transpose_block(inb.at[i], outb.at[i], 64, scale)
                fire_out(t, i).start()

        fire_out(nfull - 2, 0).wait()
        fire_out(nfull - 1, 1).wait()

        if extra:
            @pl.when(wid < extra)
            def _():
                fire_in(nfull, 0).start()
                fire_in(nfull, 0).wait()
                _transpose_block(inb.at[0], outb.at[0], 64, scale)
                fire_out(nfull, 0).start()
                fire_out(nfull, 0).wait()

        if tail:
            @pl.when(wid == extra)
            def _():
                cp = pltpu.make_async_copy(
                    tt.at[:, pl.ds(nblk * 128, tail)], int_, gs[0])
                cp.start()
                cp.wait()
                _transpose_block(int_, outt, tail // 2, scale)
                cp2 = pltpu.make_async_copy(
                    outt, tp.at[pl.ds(nblk * 64, tail // 2)], ss[0])
                cp2.start()
                cp2.wait()

    return pl.kernel(
        body,
        out_type=jax.ShapeDtypeStruct((vocab // 2, 128), jnp.float32),
        mesh=_mesh(),
        scratch_types=[
            pltpu.VMEM((2, 64, 128), jnp.float32),
            pltpu.VMEM((2, 64, 128), jnp.float32),
            pltpu.VMEM((64, 64), jnp.float32),
            pltpu.VMEM((32, 128), jnp.float32),
        ] + [pltpu.SemaphoreType.DMA] * 4,
        compiler_params=pltpu.CompilerParams(
            use_tc_tiling_on_sc=True, needs_layout_passes=False),
    )


NBUF = 4


@functools.lru_cache(maxsize=None)
def _phase2(b: int, s: int, vocab: int, d: int):
    assert d == 64 and b % NW == 0 and s % NBUF == 0
    bpw = b // NW

    def body(xt, tp, out, xblk, idxb, gbuf, tbuf, *sems):
        gsems, ssems = sems[:NBUF], sems[NBUF:]
        wid = _wid()
        b0 = wid * bpw

        pltpu.sync_copy(xt.at[:, pl.ds(b0, bpw)], xblk)

        def halve(c, bi):
            for g2 in range(bpw // L):
                sl = pl.ds(16 * g2, 16)
                idxb[bi, sl] = lax.shift_right_logical(xblk[c, sl], 1)

        def gather(c, bi):
            return pltpu.make_async_copy(
                tp.at[idxb.at[bi]], gbuf.at[bi], gsems[bi])

        def scatter(c, bi):
            return pltpu.make_async_copy(
                tbuf.at[bi], out.at[c, :, pl.ds(b0, bpw)], ssems[bi])

        halve(0, 0)
        gather(0, 0).start()
        halve(1, 1)
        gather(1, 1).start()

        @pl.loop(0, s, step=NBUF)
        def _grp(g):
            for bi in range(NBUF):
                c = g + bi
                if bi >= 2:
                    scatter(c - 2, (bi + 2) % NBUF).wait()
                else:
                    @pl.when(c >= 2)
                    def _():
                        scatter(c - 2, (bi + 2) % NBUF).wait()
                @pl.when(c + 2 < s)
                def _():
                    halve(c + 2, (bi + 2) % NBUF)
                    gather(c + 2, (bi + 2) % NBUF).start()
                gather(c, bi).wait()
                src, dst = gbuf.at[bi], tbuf.at[bi]

                @pl.loop(0, d, unroll=4)
                def _feat(j):
                    jv = jnp.full((16,), j, jnp.int32)
                    for g2 in range(bpw // L):
                        rowv = _iota16() + 16 * g2
                        par = lax.shift_left(
                            jnp.bitwise_and(
                                xblk[c, pl.ds(16 * g2, 16)], 1), 6)
                        dst[j, pl.ds(16 * g2, 16)] = plsc.load_gather(
                            src, [rowv, par + jv])

                scatter(c, bi).start()

        scatter(s - 2, (s - 2) % NBUF).wait()
        scatter(s - 1, (s - 1) % NBUF).wait()

    return pl.kernel(
        body,
        out_type=jax.ShapeDtypeStruct((s, d, b), jnp.float32),
        mesh=_mesh(),
        scratch_types=[
            pltpu.VMEM((s, bpw), jnp.int32),
            pltpu.VMEM((8, bpw), jnp.int32),
            pltpu.VMEM((NBUF, bpw, 2 * d), jnp.float32),
            pltpu.VMEM((NBUF, d, bpw), jnp.float32),
        ] + [pltpu.SemaphoreType.DMA] * (2 * NBUF),
        compiler_params=pltpu.CompilerParams(
            use_tc_tiling_on_sc=True, needs_layout_passes=False),
    )


def kernel(x, table):
    b, s = x.shape
    vocab, d = table.shape
    xt = x.astype(jnp.int32).T
    tt = table.T
    tp = _phase1(vocab, d)(tt)
    out = _phase2(b, s, vocab, d)(xt, tp)
    return out.transpose(2, 0, 1)


# speedup vs baseline: 2.6401x; 2.6401x over previous
import functools
import math

import jax
import jax.numpy as jnp
from jax import lax
from jax.experimental import pallas as pl
from jax.experimental.pallas import tpu as pltpu
from jax.experimental.pallas import tpu_sc as plsc

NC = 2
NS = 16
L = 16
NW = NC * NS


def _mesh():
    return plsc.VectorSubcoreMesh(
        core_axis_name="c", subcore_axis_name="s",
        num_cores=NC, num_subcores=NS)


def _wid():
    return lax.axis_index("s") * NC + lax.axis_index("c")


def _iota16():
    return lax.iota(jnp.int32, 16)


def _transpose_block(src, dst, rows, scale):
    rowvs = [_iota16() + 16 * k for k in range(4)]

    @plsc.parallel_loop(0, rows, unroll=4)
    def _row(r):
        for k in range(8):
            colv = jnp.full((16,), 2 * r + (1 if k >= 4 else 0), jnp.int32)
            vals = plsc.load_gather(src, [rowvs[k % 4], colv]) * scale
            dst[r, pl.ds(16 * k, 16)] = vals


@functools.lru_cache(maxsize=None)
def _phase1(vocab: int, d: int):
    assert d == 64
    nblk = vocab // 128
    tail = vocab % 128
    assert tail in (0, 64)
    nfull = nblk // NW
    extra = nblk % NW
    scale = math.sqrt(d)

    def body(tt, tp, inb, outb, int_, outt, *sems):
        gs, ss = sems[:2], sems[2:]
        wid = _wid()

        def fire_in(t, b):
            return pltpu.make_async_copy(
                tt.at[:, pl.ds((wid + NW * t) * 128, 128)], inb.at[b], gs[b])

        def fire_out(t, b):
            return pltpu.make_async_copy(
                outb.at[b], tp.at[pl.ds((wid + NW * t) * 64, 64)], ss[b])

        fire_in(0, 0).start()

        @pl.loop(0, nfull // 2)
        def _grp(g):
            for i in range(2):
                t = 2 * g + i
                @pl.when(t + 1 < nfull)
                def _():
                    fire_in(t + 1, (i + 1) % 2).start()
                fire_in(t, i).wait()
                @pl.when(t >= 2)
                def _():
                    fire_out(t - 2, i).wait()
                _transpose_block(inb.at[i], outb.at[i], 64, scale)
                fire_out(t, i).start()

        fire_out(nfull - 2, 0).wait()
        fire_out(nfull - 1, 1).wait()

        if extra:
            @pl.when(wid < extra)
            def _():
                fire_in(nfull, 0).start()
                fire_in(nfull, 0).wait()
                _transpose_block(inb.at[0], outb.at[0], 64, scale)
                fire_out(nfull, 0).start()
                fire_out(nfull, 0).wait()

        if tail:
            @pl.when(wid == extra)
            def _():
                cp = pltpu.make_async_copy(
                    tt.at[:, pl.ds(nblk * 128, tail)], int_, gs[0])
                cp.start()
                cp.wait()
                _transpose_block(int_, outt, tail // 2, scale)
                cp2 = pltpu.make_async_copy(
                    outt, tp.at[pl.ds(nblk * 64, tail // 2)], ss[0])
                cp2.start()
                cp2.wait()

    return pl.kernel(
        body,
        out_type=jax.ShapeDtypeStruct((vocab // 2, 128), jnp.float32),
        mesh=_mesh(),
        scratch_types=[
            pltpu.VMEM((2, 64, 128), jnp.float32),
            pltpu.VMEM((2, 64, 128), jnp.float32),
            pltpu.VMEM((64, 64), jnp.float32),
            pltpu.VMEM((32, 128), jnp.float32),
        ] + [pltpu.SemaphoreType.DMA] * 4,
        compiler_params=pltpu.CompilerParams(
            use_tc_tiling_on_sc=True, needs_layout_passes=False),
    )


NBUF = 4


@functools.lru_cache(maxsize=None)
def _phase2(b: int, s: int, vocab: int, d: int):
    assert d == 64 and b % NW == 0 and s % NBUF == 0
    bpw = b // NW

    def body(xt, tp, out, xblk, idxb, gbuf, tbuf, *sems):
        gsems, ssems = sems[:NBUF], sems[NBUF:]
        wid = _wid()
        b0 = wid * bpw

        pltpu.sync_copy(xt.at[:, pl.ds(b0, bpw)], xblk)

        def halve(c, bi):
            for g2 in range(bpw // L):
                sl = pl.ds(16 * g2, 16)
                idxb[bi, sl] = lax.shift_right_logical(xblk[c, sl], 1)

        def gather(c, bi):
            return pltpu.make_async_copy(
                tp.at[idxb.at[bi]], gbuf.at[bi], gsems[bi])

        def scatter(c, bi):
            return pltpu.make_async_copy(
                tbuf.at[bi], out.at[c, :, pl.ds(b0, bpw)], ssems[bi])

        halve(0, 0)
        gather(0, 0).start()
        halve(1, 1)
        gather(1, 1).start()

        @pl.loop(0, s, step=NBUF)
        def _grp(g):
            for bi in range(NBUF):
                c = g + bi
                if bi >= 2:
                    scatter(c - 2, (bi + 2) % NBUF).wait()
                else:
                    @pl.when(c >= 2)
                    def _():
                        scatter(c - 2, (bi + 2) % NBUF).wait()
                @pl.when(c + 2 < s)
                def _():
                    halve(c + 2, (bi + 2) % NBUF)
                    gather(c + 2, (bi + 2) % NBUF).start()
                gather(c, bi).wait()
                src, dst = gbuf.at[bi], tbuf.at[bi]
                cols = [
                    lax.shift_left(
                        jnp.bitwise_and(xblk[c, pl.ds(16 * g2, 16)], 1), 6)
                    for g2 in range(bpw // L)
                ]
                rows = [_iota16() + 16 * g2 for g2 in range(bpw // L)]

                @plsc.parallel_loop(0, d, unroll=4)
                def _feat(j):
                    jv = jnp.full((16,), j, jnp.int32)
                    for g2 in range(bpw // L):
                        dst[j, pl.ds(16 * g2, 16)] = plsc.load_gather(
                            src, [rows[g2], cols[g2] + jv])

                scatter(c, bi).start()

        scatter(s - 2, (s - 2) % NBUF).wait()
        scatter(s - 1, (s - 1) % NBUF).wait()

    return pl.kernel(
        body,
        out_type=jax.ShapeDtypeStruct((s, d, b), jnp.float32),
        mesh=_mesh(),
        scratch_types=[
            pltpu.VMEM((s, bpw), jnp.int32),
            pltpu.VMEM((8, bpw), jnp.int32),
            pltpu.VMEM((NBUF, bpw, 2 * d), jnp.float32),
            pltpu.VMEM((NBUF, d, bpw), jnp.float32),
        ] + [pltpu.SemaphoreType.DMA] * (2 * NBUF),
        compiler_params=pltpu.CompilerParams(
            use_tc_tiling_on_sc=True, needs_layout_passes=False),
    )


def kernel(x, table):
    b, s = x.shape
    vocab, d = table.shape
    xt = x.astype(jnp.int32).T
    tt = table.T
    tp = _phase1(vocab, d)(tt)
    out = _phase2(b, s, vocab, d)(xt, tp)
    return out.transpose(2, 0, 1)


# speedup vs baseline: 5.9322x; 2.2469x over previous
import functools
import math

import jax
import jax.numpy as jnp
from jax import lax
from jax.experimental import pallas as pl
from jax.experimental.pallas import tpu as pltpu
from jax.experimental.pallas import tpu_sc as plsc

NC = 2
NS = 16
L = 16
NW = NC * NS


def _mesh():
    return plsc.VectorSubcoreMesh(
        core_axis_name="c", subcore_axis_name="s",
        num_cores=NC, num_subcores=NS)


def _wid():
    return lax.axis_index("s") * NC + lax.axis_index("c")


def _iota16():
    return lax.iota(jnp.int32, 16)


def _transpose_block(src, dst, ncols, scale):
    iota = _iota16()
    jbs = [16 * jb + iota for jb in range(4)]

    @plsc.parallel_loop(0, 16)
    def _k(k):
        perm = jnp.bitwise_and(iota + k, 15)
        for vb in range(ncols // 16):
            v2v = perm + 16 * vb
            rv = lax.shift_right_logical(v2v, 1)
            pbit = lax.shift_left(jnp.bitwise_and(v2v, 1), 6)
            for jb in range(4):
                val = plsc.load_gather(src, [jbs[jb], v2v]) * scale
                plsc.store_scatter(dst, [rv, jbs[jb] + pbit], val)


@functools.lru_cache(maxsize=None)
def _phase1(vocab: int, d: int):
    assert d == 64
    nblk = vocab // 128
    tail = vocab % 128
    assert tail in (0, 64)
    nfull = nblk // NW
    extra = nblk % NW
    scale = math.sqrt(d)

    def body(tt, tp, inb, outb, int_, outt, *sems):
        gs, ss = sems[:2], sems[2:]
        wid = _wid()

        def fire_in(t, b):
            return pltpu.make_async_copy(
                tt.at[:, pl.ds((wid + NW * t) * 128, 128)], inb.at[b], gs[b])

        def fire_out(t, b):
            return pltpu.make_async_copy(
                outb.at[b], tp.at[pl.ds((wid + NW * t) * 64, 64)], ss[b])

        fire_in(0, 0).start()

        @pl.loop(0, nfull // 2)
        def _grp(g):
            for i in range(2):
                t = 2 * g + i
                @pl.when(t + 1 < nfull)
                def _():
                    fire_in(t + 1, (i + 1) % 2).start()
                fire_in(t, i).wait()
                @pl.when(t >= 2)
                def _():
                    fire_out(t - 2, i).wait()
                _transpose_block(inb.at[i], outb.at[i], 128, scale)
                fire_out(t, i).start()

        fire_out(nfull - 2, 0).wait()
        fire_out(nfull - 1, 1).wait()

        if extra:
            @pl.when(wid < extra)
            def _():
                fire_in(nfull, 0).start()
                fire_in(nfull, 0).wait()
                _transpose_block(inb.at[0], outb.at[0], 128, scale)
                fire_out(nfull, 0).start()
                fire_out(nfull, 0).wait()

        if tail:
            @pl.when(wid == extra)
            def _():
                cp = pltpu.make_async_copy(
                    tt.at[:, pl.ds(nblk * 128, tail)], int_, gs[0])
                cp.start()
                cp.wait()
                _transpose_block(int_, outt, tail, scale)
                cp2 = pltpu.make_async_copy(
                    outt, tp.at[pl.ds(nblk * 64, tail // 2)], ss[0])
                cp2.start()
                cp2.wait()

    return pl.kernel(
        body,
        out_type=jax.ShapeDtypeStruct((vocab // 2, 128), jnp.float32),
        mesh=_mesh(),
        scratch_types=[
            pltpu.VMEM((2, 64, 128), jnp.float32),
            pltpu.VMEM((2, 64, 128), jnp.float32),
            pltpu.VMEM((64, 64), jnp.float32),
            pltpu.VMEM((32, 128), jnp.float32),
        ] + [pltpu.SemaphoreType.DMA] * 4,
        compiler_params=pltpu.CompilerParams(
            use_tc_tiling_on_sc=True, needs_layout_passes=False),
    )


NBUF = 4


@functools.lru_cache(maxsize=None)
def _phase2(b: int, s: int, vocab: int, d: int):
    assert d == 64 and b % NW == 0 and s % NBUF == 0
    bpw = b // NW

    def body(xt, tp, out, xblk, idxb, gbuf, tbuf, *sems):
        gsems, ssems = sems[:NBUF], sems[NBUF:]
        wid = _wid()
        b0 = wid * bpw

        pltpu.sync_copy(xt.at[:, pl.ds(b0, bpw)], xblk)

        def halve(c, bi):
            for g2 in range(bpw // L):
                sl = pl.ds(16 * g2, 16)
                idxb[bi, sl] = lax.shift_right_logical(xblk[c, sl], 1)

        def gather(c, bi):
            return pltpu.make_async_copy(
                tp.at[idxb.at[bi]], gbuf.at[bi], gsems[bi])

        def scatter(c, bi):
            return pltpu.make_async_copy(
                tbuf.at[bi], out.at[c, :, pl.ds(b0, bpw)], ssems[bi])

        halve(0, 0)
        gather(0, 0).start()
        halve(1, 1)
        gather(1, 1).start()

        @pl.loop(0, s, step=NBUF)
        def _grp(g):
            for bi in range(NBUF):
                c = g + bi
                if bi >= 2:
                    scatter(c - 2, (bi + 2) % NBUF).wait()
                else:
                    @pl.when(c >= 2)
                    def _():
                        scatter(c - 2, (bi + 2) % NBUF).wait()
                @pl.when(c + 2 < s)
                def _():
                    halve(c + 2, (bi + 2) % NBUF)
                    gather(c + 2, (bi + 2) % NBUF).start()
                gather(c, bi).wait()
                src, dst = gbuf.at[bi], tbuf.at[bi]
                cols = [
                    lax.shift_left(
                        jnp.bitwise_and(xblk[c, pl.ds(16 * g2, 16)], 1), 6)
                    for g2 in range(bpw // L)
                ]
                rows = [_iota16() + 16 * g2 for g2 in range(bpw // L)]

                iota = _iota16()

                @plsc.parallel_loop(0, 16)
                def _diag(k):
                    perm = jnp.bitwise_and(iota + k, 15)
                    for h in range(d // L):
                        jv = perm + 16 * h
                        for g2 in range(bpw // L):
                            val = plsc.load_gather(
                                src, [rows[g2], cols[g2] + jv])
                            plsc.store_scatter(dst, [jv, rows[g2]], val)

                scatter(c, bi).start()

        scatter(s - 2, (s - 2) % NBUF).wait()
        scatter(s - 1, (s - 1) % NBUF).wait()

    return pl.kernel(
        body,
        out_type=jax.ShapeDtypeStruct((s, d, b), jnp.float32),
        mesh=_mesh(),
        scratch_types=[
            pltpu.VMEM((s, bpw), jnp.int32),
            pltpu.VMEM((8, bpw), jnp.int32),
            pltpu.VMEM((NBUF, bpw, 2 * d), jnp.float32),
            pltpu.VMEM((NBUF, d, bpw), jnp.float32),
        ] + [pltpu.SemaphoreType.DMA] * (2 * NBUF),
        compiler_params=pltpu.CompilerParams(
            use_tc_tiling_on_sc=True, needs_layout_passes=False),
    )


def kernel(x, table):
    b, s = x.shape
    vocab, d = table.shape
    xt = x.astype(jnp.int32).T
    tt = table.T
    tp = _phase1(vocab, d)(tt)
    out = _phase2(b, s, vocab, d)(xt, tp)
    return out.transpose(2, 0, 1)


# speedup vs baseline: 6.4772x; 1.0919x over previous
import functools
import math

import jax
import jax.numpy as jnp
from jax import lax
from jax.experimental import pallas as pl
from jax.experimental.pallas import tpu as pltpu
from jax.experimental.pallas import tpu_sc as plsc

NC = 2
NS = 16
L = 16
NW = NC * NS


def _mesh():
    return plsc.VectorSubcoreMesh(
        core_axis_name="c", subcore_axis_name="s",
        num_cores=NC, num_subcores=NS)


def _wid():
    return lax.axis_index("s") * NC + lax.axis_index("c")


def _iota16():
    return lax.iota(jnp.int32, 16)


def _transpose_block(src, dst, ncols, scale):
    iota = _iota16()
    jbs = [16 * jb + iota for jb in range(4)]

    @plsc.parallel_loop(0, 16)
    def _k(k):
        perm = jnp.bitwise_and(iota + k, 15)
        for vb in range(ncols // 16):
            v2v = perm + 16 * vb
            rv = lax.shift_right_logical(v2v, 1)
            pbit = lax.shift_left(jnp.bitwise_and(v2v, 1), 6)
            for jb in range(4):
                val = plsc.load_gather(src, [jbs[jb], v2v]) * scale
                plsc.store_scatter(dst, [rv, jbs[jb] + pbit], val)


@functools.lru_cache(maxsize=None)
def _phase1(vocab: int, d: int):
    assert d == 64
    nblk = vocab // 128
    tail = vocab % 128
    assert tail in (0, 64)
    nfull = nblk // NW
    extra = nblk % NW
    scale = math.sqrt(d)

    def body(tt, tp, inb, outb, int_, outt, *sems):
        gs, ss = sems[:2], sems[2:]
        wid = _wid()

        def fire_in(t, b):
            return pltpu.make_async_copy(
                tt.at[:, pl.ds((wid + NW * t) * 128, 128)], inb.at[b], gs[b])

        def fire_out(t, b):
            return pltpu.make_async_copy(
                outb.at[b], tp.at[pl.ds((wid + NW * t) * 64, 64)], ss[b])

        fire_in(0, 0).start()

        @pl.loop(0, nfull // 2)
        def _grp(g):
            for i in range(2):
                t = 2 * g + i
                @pl.when(t + 1 < nfull)
                def _():
                    fire_in(t + 1, (i + 1) % 2).start()
                fire_in(t, i).wait()
                @pl.when(t >= 2)
                def _():
                    fire_out(t - 2, i).wait()
                _transpose_block(inb.at[i], outb.at[i], 128, scale)
                fire_out(t, i).start()

        fire_out(nfull - 2, 0).wait()
        fire_out(nfull - 1, 1).wait()

        if extra:
            @pl.when(wid < extra)
            def _():
                fire_in(nfull, 0).start()
                fire_in(nfull, 0).wait()
                _transpose_block(inb.at[0], outb.at[0], 128, scale)
                fire_out(nfull, 0).start()
                fire_out(nfull, 0).wait()

        if tail:
            @pl.when(wid == extra)
            def _():
                cp = pltpu.make_async_copy(
                    tt.at[:, pl.ds(nblk * 128, tail)], int_, gs[0])
                cp.start()
                cp.wait()
                _transpose_block(int_, outt, tail, scale)
                cp2 = pltpu.make_async_copy(
                    outt, tp.at[pl.ds(nblk * 64, tail // 2)], ss[0])
                cp2.start()
                cp2.wait()

    return pl.kernel(
        body,
        out_type=jax.ShapeDtypeStruct((vocab // 2, 128), jnp.float32),
        mesh=_mesh(),
        scratch_types=[
            pltpu.VMEM((2, 64, 128), jnp.float32),
            pltpu.VMEM((2, 64, 128), jnp.float32),
            pltpu.VMEM((64, 64), jnp.float32),
            pltpu.VMEM((32, 128), jnp.float32),
        ] + [pltpu.SemaphoreType.DMA] * 4,
        compiler_params=pltpu.CompilerParams(
            use_tc_tiling_on_sc=True, needs_layout_passes=False),
    )


NBUF = 4


@functools.lru_cache(maxsize=None)
def _phase2(b: int, s: int, vocab: int, d: int):
    assert d == 64 and b % NW == 0 and s % NBUF == 0
    bpw = b // NW
    assert bpw == 128

    def body(xt, tl, out, xblk, gbuf, tbuf, *sems):
        gsems, ssems = sems[:NBUF], sems[NBUF:]
        wid = _wid()
        b0 = wid * bpw

        pltpu.sync_copy(xt.at[:, pl.ds(b0, bpw)], xblk)

        def gather(c, bi):
            return pltpu.make_async_copy(
                tl.at[xblk.at[c]], gbuf.at[bi], gsems[bi])

        def scatter(c, bi):
            return pltpu.make_async_copy(
                tbuf.at[bi], out.at[c, :, wid], ssems[bi])

        gather(0, 0).start()
        gather(1, 1).start()

        @pl.loop(0, s, step=NBUF)
        def _grp(g):
            for bi in range(NBUF):
                c = g + bi
                if bi >= 2:
                    scatter(c - 2, (bi + 2) % NBUF).wait()
                else:
                    @pl.when(c >= 2)
                    def _():
                        scatter(c - 2, (bi + 2) % NBUF).wait()
                @pl.when(c + 2 < s)
                def _():
                    gather(c + 2, (bi + 2) % NBUF).start()
                gather(c, bi).wait()
                src, dst = gbuf.at[bi], tbuf.at[bi]
                iota = _iota16()
                rows = [iota + 16 * g2 for g2 in range(bpw // L)]

                @plsc.parallel_loop(0, 16)
                def _diag(k):
                    perm = jnp.bitwise_and(iota + k, 15)
                    for h in range(d // L):
                        jv = perm + 16 * h
                        jb = lax.shift_right_logical(jv, 3)
                        jr = jnp.bitwise_and(jv, 7)
                        for g2 in range(bpw // L):
                            val = plsc.load_gather(src, [rows[g2], jv])
                            plsc.store_scatter(
                                dst, [jb, jr, rows[g2]], val)

                scatter(c, bi).start()

        scatter(s - 2, (s - 2) % NBUF).wait()
        scatter(s - 1, (s - 1) % NBUF).wait()

    return pl.kernel(
        body,
        out_type=jax.ShapeDtypeStruct(
            (s, d // 8, b // 128, 8, 128), jnp.float32),
        mesh=_mesh(),
        scratch_types=[
            pltpu.VMEM((s, bpw), jnp.int32),
            pltpu.VMEM((NBUF, bpw, d), jnp.float32),
            pltpu.VMEM((NBUF, d // 8, 8, bpw), jnp.float32),
        ] + [pltpu.SemaphoreType.DMA] * (2 * NBUF),
        compiler_params=pltpu.CompilerParams(
            use_tc_tiling_on_sc=False, needs_layout_passes=False),
    )


def kernel(x, table):
    b, s = x.shape
    vocab, d = table.shape
    xt = x.astype(jnp.int32).T
    tt = table.T
    tp = _phase1(vocab, d)(tt)
    tl = tp.reshape(vocab, d)
    out5 = _phase2(b, s, vocab, d)(xt, tl)
    return jnp.transpose(out5, (2, 4, 0, 1, 3)).reshape(b, s, d)
